# Initial kernel scaffold; baseline (speedup 1.0000x reference)
import jax
import jax.numpy as jnp
from jax.experimental import pallas as pl


def kernel(x, u_size, v_size):
    raise NotImplementedError("write your pallas kernel here")


# speedup vs baseline: 3.9883x; 3.9883x over previous
import functools

import jax
import jax.numpy as jnp
from jax import lax
from jax.experimental import pallas as pl
from jax.experimental.pallas import tpu as pltpu
from jax.experimental.pallas import tpu_sc as plsc

_L = 16
_UPAD = 1040
_NVREG = _UPAD // _L


def _shuf(x, lane, sh):
    return x.at[lane ^ sh].get(mode="promise_in_bounds", unique_indices=True)


def _allmax(x, lane):
    for sh in (1, 2, 4, 8):
        x = jnp.maximum(x, _shuf(x, lane, sh))
    return x


def _allmin(x, lane):
    for sh in (1, 2, 4, 8):
        x = jnp.minimum(x, _shuf(x, lane, sh))
    return x


def _greedy_body(nrows, nchunks, vb, xr, outsz, outseq, buf0, buf1, pen,
                 seqb, szb, tmpv, sem0, sem1):
    c = lax.axis_index("c")
    s = lax.axis_index("s")
    nc = plsc.get_sparse_core_info().num_cores
    wid = s * nc + c
    lane = lax.iota(jnp.int32, _L)

    szb[...] = jnp.zeros((_L,), jnp.float32)

    def row_body(r, _):
        b = wid * nrows + r

        def pen_init(j, _):
            pen[pl.ds(pl.multiple_of(j * _L, _L), _L)] = jnp.zeros(
                (_L,), jnp.float32)
            return 0
        lax.fori_loop(0, _NVREG, pen_init, 0)

        row_off = b * (nchunks * vb * _UPAD)
        cw = vb * _UPAD
        copies = [pltpu.async_copy(xr.at[pl.ds(row_off, cw)], buf0, sem0)]

        def v_step(buf, ci, vl, size_acc):
            def j_scan(j, carry):
                rmax, ridx = carry
                off = pl.multiple_of(j * _L, _L)
                xv = buf[pl.ds(pl.multiple_of(vl * _UPAD + off, _L), _L)]
                pm = pen[pl.ds(off, _L)]
                eff = jnp.where(pm > 0.5, jnp.float32(-1.0), xv)
                pred = eff > rmax
                rmax = jnp.where(pred, eff, rmax)
                ridx = jnp.where(pred, jnp.full((_L,), j, jnp.int32), ridx)
                return (rmax, ridx)

            rmax0 = jnp.full((_L,), -3.0, jnp.float32)
            ridx0 = jnp.zeros((_L,), jnp.int32)
            rmax, ridx = lax.fori_loop(0, _NVREG, j_scan, (rmax0, ridx0))

            maxv = _allmax(rmax, lane)
            gidx = ridx * _L + lane
            cand = jnp.where(rmax == maxv, gidx, jnp.int32(1 << 30))
            sel = _allmin(cand, lane)

            sel_s = sel[0]

            jsel = sel_s >> 4
            lsel = sel_s & 15
            poff = pl.multiple_of(jsel * _L, _L)
            gate = lax.select(sel_s > 0, jnp.float32(1.0), jnp.float32(0.0))
            pv = pen[pl.ds(poff, _L)]
            pen[pl.ds(poff, _L)] = jnp.where(
                lane == lsel, jnp.maximum(pv, gate), pv)

            vglob = ci * vb + vl
            qoff = pl.multiple_of((vglob >> 4) * _L, _L)
            sv = seqb[pl.ds(qoff, _L)]
            seqb[pl.ds(qoff, _L)] = jnp.where(lane == (vglob & 15), sel, sv)
            return size_acc + maxv

        size_acc = jnp.zeros((_L,), jnp.float32)
        for ci in range(nchunks):
            buf = buf0 if ci % 2 == 0 else buf1
            if ci + 1 < nchunks:
                nbuf, nsem = (buf0, sem0) if (ci + 1) % 2 == 0 else (buf1, sem1)
                copies.append(pltpu.async_copy(
                    xr.at[pl.ds(row_off + (ci + 1) * cw, cw)], nbuf, nsem))
            copies[ci].wait()
            size_acc = lax.fori_loop(
                0, vb, functools.partial(v_step, buf, ci), size_acc)

        szb[...] = jnp.where(lane == r, -size_acc, szb[...])
        pltpu.sync_copy(seqb, outseq.at[pl.ds(b * (nchunks * vb),
                                              nchunks * vb)])
        return 0

    lax.fori_loop(0, nrows, row_body, 0)
    pltpu.sync_copy(szb, outsz.at[pl.ds(wid * _L, _L)])


def kernel(x, u_size, v_size):
    B, V, U1 = x.shape
    info = plsc.get_sparse_core_info()
    nw = info.num_cores * info.num_subcores
    nrows = B // nw
    vb = 32
    nchunks = V // vb

    xp = jnp.concatenate(
        [x, jnp.full((B, V, _UPAD - U1), -1.0, jnp.float32)], axis=-1)
    xr = xp.reshape(B * V * _UPAD)

    body = functools.partial(_greedy_body, nrows, nchunks, vb)
    run = pl.kernel(
        body,
        mesh=plsc.VectorSubcoreMesh(core_axis_name="c", subcore_axis_name="s"),
        out_type=[
            jax.ShapeDtypeStruct((nw * _L,), jnp.float32),
            jax.ShapeDtypeStruct((B * V,), jnp.int32),
        ],
        scratch_types=[
            pltpu.VMEM((vb * _UPAD,), jnp.float32),
            pltpu.VMEM((vb * _UPAD,), jnp.float32),
            pltpu.VMEM((_UPAD,), jnp.float32),
            pltpu.VMEM((V,), jnp.int32),
            pltpu.VMEM((_L,), jnp.float32),
            pltpu.VMEM((_L,), jnp.int32),
            pltpu.SemaphoreType.DMA,
            pltpu.SemaphoreType.DMA,
        ],
    )
    outsz, outseq = run(xr)
    neg_size = outsz.reshape(nw, _L)[:, :nrows].reshape(B)
    return (neg_size, outseq.reshape(B, V))


# speedup vs baseline: 5.4175x; 1.3583x over previous
import functools

import jax
import jax.numpy as jnp
from jax import lax
from jax.experimental import pallas as pl
from jax.experimental.pallas import tpu as pltpu
from jax.experimental.pallas import tpu_sc as plsc

_L = 16
_UPAD = 1040
_NVREG = _UPAD // _L


def _shuf(x, lane, sh):
    return x.at[lane ^ sh].get(mode="promise_in_bounds", unique_indices=True)


def _allmax(x, lane):
    for sh in (1, 2, 4, 8):
        x = jnp.maximum(x, _shuf(x, lane, sh))
    return x


def _allmin(x, lane):
    for sh in (1, 2, 4, 8):
        x = jnp.minimum(x, _shuf(x, lane, sh))
    return x


def _greedy_body(nrows, nchunks, vb, xr, outsz, outseq, buf0, buf1, pen,
                 seqb, szb, tmpv, sem0, sem1):
    c = lax.axis_index("c")
    s = lax.axis_index("s")
    nc = plsc.get_sparse_core_info().num_cores
    wid = s * nc + c
    lane = lax.iota(jnp.int32, _L)

    szb[...] = jnp.zeros((_L,), jnp.float32)

    def row_body(r, _):
        b = wid * nrows + r

        def pen_init(j, _):
            pen[pl.ds(pl.multiple_of(j * _L, _L), _L)] = jnp.zeros(
                (_L,), jnp.float32)
            return 0
        lax.fori_loop(0, _NVREG, pen_init, 0)

        row_off = b * (nchunks * vb * _UPAD)
        cw = vb * _UPAD
        copies = [pltpu.async_copy(xr.at[pl.ds(row_off, cw)], buf0, sem0)]

        def v_step(buf, ci, vl, size_acc):
            vbase = pl.multiple_of(vl * _UPAD, _L)
            rmax = jnp.full((_L,), -3.0, jnp.float32)
            ridx = jnp.zeros((_L,), jnp.int32)
            for j in range(_NVREG):
                xv = buf[pl.ds(vbase + j * _L, _L)]
                pm = pen[pl.ds(j * _L, _L)]
                eff = jnp.where(pm > 0.5, jnp.float32(-1.0), xv)
                pred = eff > rmax
                rmax = jnp.maximum(eff, rmax)
                ridx = jnp.where(pred, jnp.full((_L,), j, jnp.int32), ridx)

            maxv = _allmax(rmax, lane)
            gidx = ridx * _L + lane
            cand = jnp.where(rmax == maxv, gidx, jnp.int32(1 << 30))
            sel = _allmin(cand, lane)

            sel_s = sel[0]

            jsel = sel_s >> 4
            lsel = sel_s & 15
            poff = pl.multiple_of(jsel * _L, _L)
            gate = lax.select(sel_s > 0, jnp.float32(1.0), jnp.float32(0.0))
            pv = pen[pl.ds(poff, _L)]
            pen[pl.ds(poff, _L)] = jnp.where(
                lane == lsel, jnp.maximum(pv, gate), pv)

            vglob = ci * vb + vl
            qoff = pl.multiple_of((vglob >> 4) * _L, _L)
            sv = seqb[pl.ds(qoff, _L)]
            seqb[pl.ds(qoff, _L)] = jnp.where(lane == (vglob & 15), sel, sv)
            return size_acc + maxv

        size_acc = jnp.zeros((_L,), jnp.float32)
        for ci in range(nchunks):
            buf = buf0 if ci % 2 == 0 else buf1
            if ci + 1 < nchunks:
                nbuf, nsem = (buf0, sem0) if (ci + 1) % 2 == 0 else (buf1, sem1)
                copies.append(pltpu.async_copy(
                    xr.at[pl.ds(row_off + (ci + 1) * cw, cw)], nbuf, nsem))
            copies[ci].wait()
            size_acc = lax.fori_loop(
                0, vb, functools.partial(v_step, buf, ci), size_acc)

        szb[...] = jnp.where(lane == r, -size_acc, szb[...])
        pltpu.sync_copy(seqb, outseq.at[pl.ds(b * (nchunks * vb),
                                              nchunks * vb)])
        return 0

    lax.fori_loop(0, nrows, row_body, 0)
    pltpu.sync_copy(szb, outsz.at[pl.ds(wid * _L, _L)])


def kernel(x, u_size, v_size):
    B, V, U1 = x.shape
    info = plsc.get_sparse_core_info()
    nw = info.num_cores * info.num_subcores
    nrows = B // nw
    vb = 32
    nchunks = V // vb

    xp = jnp.concatenate(
        [x, jnp.full((B, V, _UPAD - U1), -1.0, jnp.float32)], axis=-1)
    xr = xp.reshape(B * V * _UPAD)

    body = functools.partial(_greedy_body, nrows, nchunks, vb)
    run = pl.kernel(
        body,
        mesh=plsc.VectorSubcoreMesh(core_axis_name="c", subcore_axis_name="s"),
        out_type=[
            jax.ShapeDtypeStruct((nw * _L,), jnp.float32),
            jax.ShapeDtypeStruct((B * V,), jnp.int32),
        ],
        scratch_types=[
            pltpu.VMEM((vb * _UPAD,), jnp.float32),
            pltpu.VMEM((vb * _UPAD,), jnp.float32),
            pltpu.VMEM((_UPAD,), jnp.float32),
            pltpu.VMEM((V,), jnp.int32),
            pltpu.VMEM((_L,), jnp.float32),
            pltpu.VMEM((_L,), jnp.int32),
            pltpu.SemaphoreType.DMA,
            pltpu.SemaphoreType.DMA,
        ],
    )
    outsz, outseq = run(xr)
    neg_size = outsz.reshape(nw, _L)[:, :nrows].reshape(B)
    return (neg_size, outseq.reshape(B, V))


# speedup vs baseline: 6.0111x; 1.1096x over previous
import functools

import jax
import jax.numpy as jnp
from jax import lax
from jax.experimental import pallas as pl
from jax.experimental.pallas import tpu as pltpu
from jax.experimental.pallas import tpu_sc as plsc

_L = 16
_UPAD = 1040
_NVREG = _UPAD // _L


def _shuf(x, lane, sh):
    return x.at[lane ^ sh].get(mode="promise_in_bounds", unique_indices=True)


def _allmax(x, lane):
    for sh in (1, 2, 4, 8):
        x = jnp.maximum(x, _shuf(x, lane, sh))
    return x


def _allmin(x, lane):
    for sh in (1, 2, 4, 8):
        x = jnp.minimum(x, _shuf(x, lane, sh))
    return x


def _greedy_body(nrows, nchunks, vb, xr, outsz, outseq, buf0, buf1, pen,
                 seqb, szb, tmpv, sem0, sem1):
    c = lax.axis_index("c")
    s = lax.axis_index("s")
    nc = plsc.get_sparse_core_info().num_cores
    wid = s * nc + c
    lane = lax.iota(jnp.int32, _L)

    szb[...] = jnp.zeros((_L,), jnp.float32)

    def row_body(r, _):
        b = wid * nrows + r

        def pen_init(j, _):
            pen[pl.ds(pl.multiple_of(j * _L, _L), _L)] = jnp.zeros(
                (_L,), jnp.float32)
            return 0
        lax.fori_loop(0, _NVREG, pen_init, 0)

        row_off = b * (nchunks * vb * _UPAD)
        cw = vb * _UPAD
        copies = [pltpu.async_copy(xr.at[pl.ds(row_off, cw)], buf0, sem0)]

        def v_step(buf, ci, vl, carry):
            size_acc, seqvec = carry
            vbase = pl.multiple_of(vl * _UPAD, _L)
            nacc = 4
            bounds = [0, 17, 34, 51, _NVREG]
            accs = []
            for a in range(nacc):
                rmax = jnp.full((_L,), -3.0, jnp.float32)
                ridx = jnp.zeros((_L,), jnp.int32)
                for j in range(bounds[a], bounds[a + 1]):
                    xv = buf[pl.ds(vbase + j * _L, _L)]
                    pm = pen[pl.ds(j * _L, _L)]
                    eff = jnp.where(pm > 0.5, jnp.float32(-1.0), xv)
                    pred = eff > rmax
                    rmax = jnp.maximum(eff, rmax)
                    ridx = jnp.where(pred, jnp.full((_L,), j, jnp.int32),
                                     ridx)
                accs.append((rmax, ridx))
            while len(accs) > 1:
                merged = []
                for (ra, ia), (rb, ib) in zip(accs[0::2], accs[1::2]):
                    pred = rb > ra
                    merged.append((jnp.maximum(ra, rb),
                                   jnp.where(pred, ib, ia)))
                accs = merged
            rmax, ridx = accs[0]

            maxv = _allmax(rmax, lane)
            gidx = ridx * _L + lane
            cand = jnp.where(rmax == maxv, gidx, jnp.int32(1 << 30))
            sel = _allmin(cand, lane)

            sel_s = sel[0]

            jsel = sel_s >> 4
            lsel = sel_s & 15
            poff = pl.multiple_of(jsel * _L, _L)
            gate = lax.select(sel_s > 0, jnp.float32(1.0), jnp.float32(0.0))
            pv = pen[pl.ds(poff, _L)]
            pen[pl.ds(poff, _L)] = jnp.where(
                lane == lsel, jnp.maximum(pv, gate), pv)

            vglob = ci * vb + vl
            seqvec = jnp.where(lane == (vglob & 15), sel, seqvec)

            @pl.when((vglob & 15) == 15)
            def _():
                seqb[pl.ds(pl.multiple_of((vglob >> 4) * _L, _L),
                           _L)] = seqvec

            return (size_acc + maxv, seqvec)

        carry = (jnp.zeros((_L,), jnp.float32),
                 jnp.zeros((_L,), jnp.int32))
        for ci in range(nchunks):
            buf = buf0 if ci % 2 == 0 else buf1
            if ci + 1 < nchunks:
                nbuf, nsem = (buf0, sem0) if (ci + 1) % 2 == 0 else (buf1, sem1)
                copies.append(pltpu.async_copy(
                    xr.at[pl.ds(row_off + (ci + 1) * cw, cw)], nbuf, nsem))
            copies[ci].wait()
            carry = lax.fori_loop(
                0, vb, functools.partial(v_step, buf, ci), carry)
        size_acc = carry[0]

        szb[...] = jnp.where(lane == r, -size_acc, szb[...])
        pltpu.sync_copy(seqb, outseq.at[pl.ds(b * (nchunks * vb),
                                              nchunks * vb)])
        return 0

    lax.fori_loop(0, nrows, row_body, 0)
    pltpu.sync_copy(szb, outsz.at[pl.ds(wid * _L, _L)])


def kernel(x, u_size, v_size):
    B, V, U1 = x.shape
    info = plsc.get_sparse_core_info()
    nw = info.num_cores * info.num_subcores
    nrows = B // nw
    vb = 32
    nchunks = V // vb

    xp = jnp.concatenate(
        [x, jnp.full((B, V, _UPAD - U1), -1.0, jnp.float32)], axis=-1)
    xr = xp.reshape(B * V * _UPAD)

    body = functools.partial(_greedy_body, nrows, nchunks, vb)
    run = pl.kernel(
        body,
        mesh=plsc.VectorSubcoreMesh(core_axis_name="c", subcore_axis_name="s"),
        out_type=[
            jax.ShapeDtypeStruct((nw * _L,), jnp.float32),
            jax.ShapeDtypeStruct((B * V,), jnp.int32),
        ],
        scratch_types=[
            pltpu.VMEM((vb * _UPAD,), jnp.float32),
            pltpu.VMEM((vb * _UPAD,), jnp.float32),
            pltpu.VMEM((_UPAD,), jnp.float32),
            pltpu.VMEM((V,), jnp.int32),
            pltpu.VMEM((_L,), jnp.float32),
            pltpu.VMEM((_L,), jnp.int32),
            pltpu.SemaphoreType.DMA,
            pltpu.SemaphoreType.DMA,
        ],
    )
    outsz, outseq = run(xr)
    neg_size = outsz.reshape(nw, _L)[:, :nrows].reshape(B)
    return (neg_size, outseq.reshape(B, V))


# speedup vs baseline: 6.7347x; 1.1204x over previous
import functools

import jax
import jax.numpy as jnp
from jax import lax
from jax.experimental import pallas as pl
from jax.experimental.pallas import tpu as pltpu
from jax.experimental.pallas import tpu_sc as plsc

_L = 16
_U1 = 1025
_NVREG = 65


def _shuf(x, lane, sh):
    return x.at[lane ^ sh].get(mode="promise_in_bounds", unique_indices=True)


def _allmax(x, lane):
    for sh in (1, 2, 4, 8):
        x = jnp.maximum(x, _shuf(x, lane, sh))
    return x


def _allmin(x, lane):
    for sh in (1, 2, 4, 8):
        x = jnp.minimum(x, _shuf(x, lane, sh))
    return x


def _greedy_body(nrows, nchunks, vb, xr, outsz, outseq, buf0, buf1, pen,
                 seqb, szb, tmpv, sem0, sem1):
    c = lax.axis_index("c")
    s = lax.axis_index("s")
    nc = plsc.get_sparse_core_info().num_cores
    wid = s * nc + c
    lane = lax.iota(jnp.int32, _L)

    szb[...] = jnp.zeros((_L,), jnp.float32)

    def row_body(r, _):
        b = wid * nrows + r

        def pen_init(j, _):
            pen[pl.ds(pl.multiple_of(j * _L, _L), _L)] = jnp.zeros(
                (_L,), jnp.float32)
            return 0
        lax.fori_loop(0, _NVREG, pen_init, 0)

        cw = vb * _U1
        row_off = b * (nchunks * cw)
        copies = [pltpu.async_copy(
            xr.at[pl.ds(row_off, cw)], buf0.at[pl.ds(0, cw)], sem0)]

        def v_step(buf, ci, vl, carry):
            size_acc, seqvec = carry
            vbase = vl * _U1
            nacc = 4
            bounds = [0, 17, 34, 51, _NVREG]
            accs = []
            for a in range(nacc):
                rmax = jnp.full((_L,), -3.0, jnp.float32)
                ridx = jnp.zeros((_L,), jnp.int32)
                for j in range(bounds[a], bounds[a + 1]):
                    xv = buf[pl.ds(vbase + j * _L, _L)]
                    pm = pen[pl.ds(j * _L, _L)]
                    eff = jnp.where(pm > 0.5, jnp.float32(-1.0), xv)
                    if j == _NVREG - 1:
                        eff = jnp.where(lane == 0, eff, jnp.float32(-1.0))
                    pred = eff > rmax
                    rmax = jnp.maximum(eff, rmax)
                    ridx = jnp.where(pred, jnp.full((_L,), j, jnp.int32),
                                     ridx)
                accs.append((rmax, ridx))
            while len(accs) > 1:
                merged = []
                for (ra, ia), (rb, ib) in zip(accs[0::2], accs[1::2]):
                    pred = rb > ra
                    merged.append((jnp.maximum(ra, rb),
                                   jnp.where(pred, ib, ia)))
                accs = merged
            rmax, ridx = accs[0]

            maxv = _allmax(rmax, lane)
            gidx = ridx * _L + lane
            cand = jnp.where(rmax == maxv, gidx, jnp.int32(1 << 30))
            sel = _allmin(cand, lane)

            sel_s = sel[0]

            jsel = sel_s >> 4
            lsel = sel_s & 15
            poff = pl.multiple_of(jsel * _L, _L)
            gate = lax.select(sel_s > 0, jnp.float32(1.0), jnp.float32(0.0))
            pv = pen[pl.ds(poff, _L)]
            pen[pl.ds(poff, _L)] = jnp.where(
                lane == lsel, jnp.maximum(pv, gate), pv)

            vglob = ci * vb + vl
            seqvec = jnp.where(lane == (vglob & 15), sel, seqvec)

            @pl.when((vglob & 15) == 15)
            def _():
                seqb[pl.ds(pl.multiple_of((vglob >> 4) * _L, _L),
                           _L)] = seqvec

            return (size_acc + maxv, seqvec)

        carry = (jnp.zeros((_L,), jnp.float32),
                 jnp.zeros((_L,), jnp.int32))
        for ci in range(nchunks):
            buf = buf0 if ci % 2 == 0 else buf1
            if ci + 1 < nchunks:
                nbuf, nsem = (buf0, sem0) if (ci + 1) % 2 == 0 else (buf1, sem1)
                copies.append(pltpu.async_copy(
                    xr.at[pl.ds(row_off + (ci + 1) * cw, cw)],
                    nbuf.at[pl.ds(0, cw)], nsem))
            copies[ci].wait()
            carry = lax.fori_loop(
                0, vb, functools.partial(v_step, buf, ci), carry)
        size_acc = carry[0]

        szb[...] = jnp.where(lane == r, -size_acc, szb[...])
        pltpu.sync_copy(seqb, outseq.at[pl.ds(b * (nchunks * vb),
                                              nchunks * vb)])
        return 0

    lax.fori_loop(0, nrows, row_body, 0)
    pltpu.sync_copy(szb, outsz.at[pl.ds(wid * _L, _L)])


def kernel(x, u_size, v_size):
    B, V, U1 = x.shape
    info = plsc.get_sparse_core_info()
    nw = info.num_cores * info.num_subcores
    nrows = B // nw
    vb = 32
    nchunks = V // vb

    xr = x.reshape(B * V * U1)

    body = functools.partial(_greedy_body, nrows, nchunks, vb)
    run = pl.kernel(
        body,
        mesh=plsc.VectorSubcoreMesh(core_axis_name="c", subcore_axis_name="s"),
        out_type=[
            jax.ShapeDtypeStruct((nw * _L,), jnp.float32),
            jax.ShapeDtypeStruct((B * V,), jnp.int32),
        ],
        scratch_types=[
            pltpu.VMEM((vb * _U1 + _L,), jnp.float32),
            pltpu.VMEM((vb * _U1 + _L,), jnp.float32),
            pltpu.VMEM((_NVREG * _L,), jnp.float32),
            pltpu.VMEM((V,), jnp.int32),
            pltpu.VMEM((_L,), jnp.float32),
            pltpu.VMEM((_L,), jnp.int32),
            pltpu.SemaphoreType.DMA,
            pltpu.SemaphoreType.DMA,
        ],
    )
    outsz, outseq = run(xr)
    neg_size = outsz.reshape(nw, _L)[:, :nrows].reshape(B)
    return (neg_size, outseq.reshape(B, V))


# speedup vs baseline: 10.8764x; 1.6150x over previous
import functools

import jax
import jax.numpy as jnp
from jax import lax
from jax.experimental import pallas as pl
from jax.experimental.pallas import tpu as pltpu
from jax.experimental.pallas import tpu_sc as plsc

_L = 16
_U1 = 1025
_NVREG = 64


def _shuf(x, idx):
    return x.at[idx].get(mode="promise_in_bounds", unique_indices=True)


def _greedy_body(nrows, nchunks, vb, xr, tl, outsz, outseq, buf0, buf1, pen,
                 tailb, seqb, szb, sem0, sem1):
    c = lax.axis_index("c")
    s = lax.axis_index("s")
    nc = plsc.get_sparse_core_info().num_cores
    wid = s * nc + c
    lane = lax.iota(jnp.int32, _L)
    nv = nchunks * vb

    szb[...] = jnp.zeros((_L,), jnp.float32)

    def row_body(r, _):
        b = wid * nrows + r

        def pen_init(j, _):
            pen[pl.ds(pl.multiple_of(j * _L, _L), _L)] = jnp.zeros(
                (_L,), jnp.float32)
            return 0
        lax.fori_loop(0, _NVREG + 1, pen_init, 0)

        pltpu.sync_copy(tl.at[pl.ds(b * nv, nv)], tailb)
        copies = [pltpu.async_copy(xr.at[b, pl.ds(0, vb), :], buf0, sem0)]

        def v_step(buf, ci, vl, carry):
            size_acc, seqvec = carry
            bounds = [0, 16, 32, 48, 64]
            accs = []
            for a in range(4):
                rmax = jnp.full((_L,), -3.0, jnp.float32)
                ridx = jnp.zeros((_L,), jnp.int32)
                for j in range(bounds[a], bounds[a + 1]):
                    xv = buf[vl, pl.ds(j * _L, _L)]
                    pm = pen[pl.ds(j * _L, _L)]
                    eff = jnp.where(pm > 0.5, jnp.float32(-1.0), xv)
                    pred = eff > rmax
                    rmax = jnp.maximum(eff, rmax)
                    ridx = jnp.where(pred, jnp.full((_L,), j, jnp.int32),
                                     ridx)
                accs.append((rmax, ridx))
            while len(accs) > 1:
                merged = []
                for (ra, ia), (rb, ib) in zip(accs[0::2], accs[1::2]):
                    pred = rb > ra
                    merged.append((jnp.maximum(ra, rb),
                                   jnp.where(pred, ib, ia)))
                accs = merged
            rmax, ridx = accs[0]
            gidx = ridx * _L + lane

            vglob = ci * vb + vl
            tvec = tailb[pl.ds(pl.multiple_of((vglob >> 4) * _L, _L), _L)]
            tval = _shuf(tvec, jnp.full((_L,), vglob & 15, jnp.int32))
            pv1024 = pen[pl.ds(_NVREG * _L, _L)]
            efft = jnp.where(pv1024 > 0.5, jnp.float32(-1.0), tval)
            efft = jnp.where(lane == 0, efft, jnp.float32(-1.0))
            predt = efft > rmax
            rmax = jnp.maximum(efft, rmax)
            gidx = jnp.where(predt, lane + _NVREG * _L, gidx)

            maxv = rmax
            for sh in (1, 2, 4, 8):
                maxv = jnp.maximum(maxv, _shuf(maxv, lane ^ sh))
            cand = jnp.where(rmax == maxv, gidx, jnp.int32(1 << 30))
            sel = cand
            for sh in (1, 2, 4, 8):
                sel = jnp.minimum(sel, _shuf(sel, lane ^ sh))
            sel_s = sel[0]

            jsel = sel_s >> 4
            lsel = sel_s & 15
            gate = lax.select(sel_s > 0, jnp.float32(1.0), jnp.float32(0.0))
            poff = pl.multiple_of(jsel * _L, _L)
            pv = pen[pl.ds(poff, _L)]
            pen[pl.ds(poff, _L)] = jnp.where(
                lane == lsel, jnp.maximum(pv, gate), pv)

            seqvec = jnp.where(lane == (vglob & 15), sel, seqvec)

            @pl.when((vglob & 15) == 15)
            def _():
                seqb[pl.ds(pl.multiple_of((vglob >> 4) * _L, _L),
                           _L)] = seqvec

            return (size_acc + maxv, seqvec)

        carry = (jnp.zeros((_L,), jnp.float32),
                 jnp.zeros((_L,), jnp.int32))
        for ci in range(nchunks):
            buf = buf0 if ci % 2 == 0 else buf1
            if ci + 1 < nchunks:
                nbuf, nsem = (buf0, sem0) if (ci + 1) % 2 == 0 else (buf1, sem1)
                copies.append(pltpu.async_copy(
                    xr.at[b, pl.ds((ci + 1) * vb, vb), :], nbuf, nsem))
            copies[ci].wait()
            carry = lax.fori_loop(
                0, vb, functools.partial(v_step, buf, ci), carry)
        size_acc = carry[0]

        szb[...] = jnp.where(lane == r, -size_acc, szb[...])
        pltpu.sync_copy(seqb, outseq.at[pl.ds(b * nv, nv)])
        return 0

    lax.fori_loop(0, nrows, row_body, 0)
    pltpu.sync_copy(szb, outsz.at[pl.ds(wid * _L, _L)])


def kernel(x, u_size, v_size):
    B, V, U1 = x.shape
    info = plsc.get_sparse_core_info()
    nw = info.num_cores * info.num_subcores
    nrows = B // nw
    vb = 32
    nchunks = V // vb

    tl = x[:, :, U1 - 1].reshape(B * V)

    body = functools.partial(_greedy_body, nrows, nchunks, vb)
    run = pl.kernel(
        body,
        mesh=plsc.VectorSubcoreMesh(core_axis_name="c", subcore_axis_name="s"),
        out_type=[
            jax.ShapeDtypeStruct((nw * _L,), jnp.float32),
            jax.ShapeDtypeStruct((B * V,), jnp.int32),
        ],
        scratch_types=[
            pltpu.VMEM((vb, _U1), jnp.float32),
            pltpu.VMEM((vb, _U1), jnp.float32),
            pltpu.VMEM(((_NVREG + 1) * _L,), jnp.float32),
            pltpu.VMEM((V,), jnp.float32),
            pltpu.VMEM((V,), jnp.int32),
            pltpu.VMEM((_L,), jnp.float32),
            pltpu.SemaphoreType.DMA,
            pltpu.SemaphoreType.DMA,
        ],
        compiler_params=pltpu.CompilerParams(use_tc_tiling_on_sc=True),
    )
    outsz, outseq = run(x, tl)
    neg_size = outsz.reshape(nw, _L)[:, :nrows].reshape(B)
    return (neg_size, outseq.reshape(B, V))
